# Initial kernel scaffold; baseline (speedup 1.0000x reference)
import jax
import jax.numpy as jnp
from jax.experimental import pallas as pl


def kernel(h, edge_index):
    raise NotImplementedError("write your pallas kernel here")


# speedup vs baseline: 3.3270x; 3.3270x over previous
import jax
import jax.numpy as jnp
from jax import lax
from jax.experimental import pallas as pl
from jax.experimental.pallas import tpu as pltpu, tpu_sc as plsc

_NUM_NODES = 10000
_DIM = 128
_NUM_EDGES = 320000

_info = plsc.get_sparse_core_info()
_NC, _NS, _L = _info.num_cores, _info.num_subcores, _info.num_lanes
_NW = _NC * _NS
_EPW = _NUM_EDGES // _NW
_C = 80
_NCHUNK = _EPW // _C
_G = _C // _L
_NSLC = _DIM // _L


def _sc_body(h_hbm, src_hbm, dst_hbm, out_hbm,
             sidx, didx, srows, drows, obuf, sem_s, sem_d):
    wid = lax.axis_index("s") * _NC + lax.axis_index("c")
    wbase = wid * _EPW

    def chunk_body(c, carry):
        base = wbase + c * _C
        pltpu.sync_copy(src_hbm.at[pl.ds(base, _C)], sidx)
        pltpu.sync_copy(dst_hbm.at[pl.ds(base, _C)], didx)
        cp_s = pltpu.async_copy(h_hbm.at[sidx], srows, sem_s)
        cp_d = pltpu.async_copy(h_hbm.at[didx], drows, sem_d)
        cp_s.wait()
        cp_d.wait()

        lanes = lax.iota(jnp.int32, _L)

        def group_body(g, gcarry):
            eb = g * _L
            dots = jnp.zeros((_L,), jnp.float32)
            for j in range(_L):
                e = eb + j
                acc = srows[e, pl.ds(0, _L)] * drows[e, pl.ds(0, _L)]
                for s in range(1, _NSLC):
                    acc = acc + srows[e, pl.ds(s * _L, _L)] * drows[e, pl.ds(s * _L, _L)]
                part = [acc[i] for i in range(_L)]
                while len(part) > 1:
                    part = [part[i] + part[i + 1] for i in range(0, len(part), 2)]
                dots = jnp.where(lanes == j, part[0], dots)
            obuf[pl.ds(eb, _L)] = 1.0 / (1.0 + jnp.exp(-dots))
            return gcarry

        lax.fori_loop(0, _G, group_body, 0)

        pltpu.sync_copy(obuf, out_hbm.at[pl.ds(base, _C)])
        return carry

    lax.fori_loop(0, _NCHUNK, chunk_body, 0)


def kernel(h, edge_index):
    src = edge_index[0].astype(jnp.int32)
    dst = edge_index[1].astype(jnp.int32)
    mesh = plsc.VectorSubcoreMesh(core_axis_name="c", subcore_axis_name="s")
    k = pl.kernel(
        _sc_body,
        out_type=jax.ShapeDtypeStruct((_NUM_EDGES,), jnp.float32),
        mesh=mesh,
        scratch_types=[
            pltpu.VMEM((_C,), jnp.int32),
            pltpu.VMEM((_C,), jnp.int32),
            pltpu.VMEM((_C, _DIM), jnp.float32),
            pltpu.VMEM((_C, _DIM), jnp.float32),
            pltpu.VMEM((_C,), jnp.float32),
            pltpu.SemaphoreType.DMA,
            pltpu.SemaphoreType.DMA,
        ],
    )
    return k(h, src, dst)


# speedup vs baseline: 6.6064x; 1.9857x over previous
import jax
import jax.numpy as jnp
from jax import lax
from jax.experimental import pallas as pl
from jax.experimental.pallas import tpu as pltpu, tpu_sc as plsc

_NUM_NODES = 10000
_DIM = 128
_NUM_EDGES = 320000

_info = plsc.get_sparse_core_info()
_NC, _NS, _L = _info.num_cores, _info.num_subcores, _info.num_lanes
_NW = _NC * _NS
_EPW = _NUM_EDGES // _NW
_C = 80
_NCHUNK = _EPW // _C
_G = _C // _L
_NSLC = _DIM // _L
_NBUF = 5


def _sc_body(h_hbm, src_hbm, dst_hbm, out_hbm,
             sidx, didx, srows, drows, obuf, gsem, osem):
    wid = lax.axis_index("s") * _NC + lax.axis_index("c")
    wbase = wid * _EPW

    pltpu.sync_copy(src_hbm.at[pl.ds(wbase, _EPW)], sidx)
    pltpu.sync_copy(dst_hbm.at[pl.ds(wbase, _EPW)], didx)

    def issue_gather(c, k):
        pltpu.async_copy(h_hbm.at[sidx.at[pl.ds(c * _C, _C)]],
                         srows.at[k], gsem.at[k, 0])
        pltpu.async_copy(h_hbm.at[didx.at[pl.ds(c * _C, _C)]],
                         drows.at[k], gsem.at[k, 1])

    def wait_gather(k):
        pltpu.make_async_copy(h_hbm.at[sidx.at[pl.ds(0, _C)]],
                              srows.at[k], gsem.at[k, 0]).wait()
        pltpu.make_async_copy(h_hbm.at[didx.at[pl.ds(0, _C)]],
                              drows.at[k], gsem.at[k, 1]).wait()

    for k in range(_NBUF):
        issue_gather(k, k)

    lanes = lax.iota(jnp.int32, _L)

    def iter_body(i, carry):
        for k in range(_NBUF):
            c = i * _NBUF + k
            wait_gather(k)
            sr = srows.at[k]
            dr = drows.at[k]

            @pl.when(i > 0)
            def _():
                pltpu.make_async_copy(
                    obuf.at[k], out_hbm.at[pl.ds(wbase, _C)], osem.at[k]
                ).wait()

            def group_body(g, gcarry):
                eb = g * _L
                dots = jnp.zeros((_L,), jnp.float32)
                for j in range(_L):
                    e = eb + j
                    acc = sr[e, pl.ds(0, _L)] * dr[e, pl.ds(0, _L)]
                    for s in range(1, _NSLC):
                        acc = acc + sr[e, pl.ds(s * _L, _L)] * dr[e, pl.ds(s * _L, _L)]
                    part = [acc[m] for m in range(_L)]
                    while len(part) > 1:
                        part = [part[m] + part[m + 1] for m in range(0, len(part), 2)]
                    dots = jnp.where(lanes == j, part[0], dots)
                obuf[k, pl.ds(eb, _L)] = 1.0 / (1.0 + jnp.exp(-dots))
                return gcarry

            lax.fori_loop(0, _G, group_body, 0)

            pltpu.async_copy(obuf.at[k],
                             out_hbm.at[pl.ds(wbase + c * _C, _C)], osem.at[k])

            @pl.when(c + _NBUF < _NCHUNK)
            def _():
                issue_gather(c + _NBUF, k)
        return carry

    lax.fori_loop(0, _NCHUNK // _NBUF, iter_body, 0)

    for k in range(_NBUF):
        pltpu.make_async_copy(obuf.at[k], out_hbm.at[pl.ds(wbase, _C)],
                              osem.at[k]).wait()


def kernel(h, edge_index):
    src = edge_index[0].astype(jnp.int32)
    dst = edge_index[1].astype(jnp.int32)
    mesh = plsc.VectorSubcoreMesh(core_axis_name="c", subcore_axis_name="s")
    k = pl.kernel(
        _sc_body,
        out_type=jax.ShapeDtypeStruct((_NUM_EDGES,), jnp.float32),
        mesh=mesh,
        scratch_types=[
            pltpu.VMEM((_EPW,), jnp.int32),
            pltpu.VMEM((_EPW,), jnp.int32),
            pltpu.VMEM((_NBUF, _C, _DIM), jnp.float32),
            pltpu.VMEM((_NBUF, _C, _DIM), jnp.float32),
            pltpu.VMEM((_NBUF, _C), jnp.float32),
            pltpu.SemaphoreType.DMA((_NBUF, 2)),
            pltpu.SemaphoreType.DMA((_NBUF,)),
        ],
    )
    return k(h, src, dst)


# speedup vs baseline: 7.3578x; 1.1137x over previous
import jax
import jax.numpy as jnp
from jax import lax
from jax.experimental import pallas as pl
from jax.experimental.pallas import tpu as pltpu, tpu_sc as plsc

_NUM_NODES = 10000
_DIM = 128
_NUM_EDGES = 320000

_info = plsc.get_sparse_core_info()
_NC, _NS, _L = _info.num_cores, _info.num_subcores, _info.num_lanes
_NW = _NC * _NS
_EPW = _NUM_EDGES // _NW
_C = 80
_NCHUNK = _EPW // _C
_G = _C // _L
_NSLC = _DIM // _L
_NBUF = 5


def _sc_body(h_hbm, src_hbm, dst_hbm, out_hbm,
             sidx, didx, srows, drows, obuf, sbuf, gsem, osem):
    wid = lax.axis_index("s") * _NC + lax.axis_index("c")
    wbase = wid * _EPW

    pltpu.sync_copy(src_hbm.at[pl.ds(wbase, _EPW)], sidx)
    pltpu.sync_copy(dst_hbm.at[pl.ds(wbase, _EPW)], didx)

    def issue_gather(c, k):
        pltpu.async_copy(h_hbm.at[sidx.at[pl.ds(c * _C, _C)]],
                         srows.at[k], gsem.at[k, 0])
        pltpu.async_copy(h_hbm.at[didx.at[pl.ds(c * _C, _C)]],
                         drows.at[k], gsem.at[k, 1])

    def wait_gather(k):
        pltpu.make_async_copy(h_hbm.at[sidx.at[pl.ds(0, _C)]],
                              srows.at[k], gsem.at[k, 0]).wait()
        pltpu.make_async_copy(h_hbm.at[didx.at[pl.ds(0, _C)]],
                              drows.at[k], gsem.at[k, 1]).wait()

    for k in range(_NBUF):
        issue_gather(k, k)

    lanes = lax.iota(jnp.int32, _L)

    def iter_body(i, carry):
        for k in range(_NBUF):
            c = i * _NBUF + k
            wait_gather(k)
            sr = srows.at[k]
            dr = drows.at[k]

            @pl.when(i > 0)
            def _():
                pltpu.make_async_copy(
                    obuf.at[k], out_hbm.at[pl.ds(wbase, _C)], osem.at[k]
                ).wait()

            def group_body(g, gcarry):
                eb = g * _L
                dots = jnp.zeros((_L,), jnp.float32)
                for j in range(_L):
                    e = eb + j
                    acc = sr[e, pl.ds(0, _L)] * dr[e, pl.ds(0, _L)]
                    for s in range(1, _NSLC):
                        acc = acc + sr[e, pl.ds(s * _L, _L)] * dr[e, pl.ds(s * _L, _L)]
                    sb = j * 32
                    sbuf[pl.ds(sb, _L)] = acc
                    half = acc + sbuf[pl.ds(sb + 8, _L)]
                    d = half[0]
                    for m in range(1, 8):
                        d = d + half[m]
                    dots = jnp.where(lanes == j, d, dots)
                obuf[k, pl.ds(eb, _L)] = 1.0 / (1.0 + jnp.exp(-dots))
                return gcarry

            lax.fori_loop(0, _G, group_body, 0)

            pltpu.async_copy(obuf.at[k],
                             out_hbm.at[pl.ds(wbase + c * _C, _C)], osem.at[k])

            @pl.when(c + _NBUF < _NCHUNK)
            def _():
                issue_gather(c + _NBUF, k)
        return carry

    lax.fori_loop(0, _NCHUNK // _NBUF, iter_body, 0)

    for k in range(_NBUF):
        pltpu.make_async_copy(obuf.at[k], out_hbm.at[pl.ds(wbase, _C)],
                              osem.at[k]).wait()


def kernel(h, edge_index):
    src = edge_index[0].astype(jnp.int32)
    dst = edge_index[1].astype(jnp.int32)
    mesh = plsc.VectorSubcoreMesh(core_axis_name="c", subcore_axis_name="s")
    k = pl.kernel(
        _sc_body,
        out_type=jax.ShapeDtypeStruct((_NUM_EDGES,), jnp.float32),
        mesh=mesh,
        scratch_types=[
            pltpu.VMEM((_EPW,), jnp.int32),
            pltpu.VMEM((_EPW,), jnp.int32),
            pltpu.VMEM((_NBUF, _C, _DIM), jnp.float32),
            pltpu.VMEM((_NBUF, _C, _DIM), jnp.float32),
            pltpu.VMEM((_NBUF, _C), jnp.float32),
            pltpu.VMEM((_L * 32,), jnp.float32),
            pltpu.SemaphoreType.DMA((_NBUF, 2)),
            pltpu.SemaphoreType.DMA((_NBUF,)),
        ],
    )
    return k(h, src, dst)
